# Initial kernel scaffold; baseline (speedup 1.0000x reference)
import jax
import jax.numpy as jnp
from jax.experimental import pallas as pl


def kernel(x, token_lora_indices, base_gate_w, base_up_w, base_down_w, retain_gate_stacked, retain_up_stacked, retain_down_stacked, forget_gate_stacked, forget_up_stacked, forget_down_stacked, scales):
    raise NotImplementedError("write your pallas kernel here")


# speedup vs baseline: 6.1341x; 6.1341x over previous
import functools

import jax
import jax.numpy as jnp
from jax.experimental import pallas as pl
from jax.experimental.pallas import tpu as pltpu

_HID = 1024
_INTER = 4096
_E = 64
_NR = 32
_NF = 32
_T = 2048

_CB = 256
_NB_BASE = _INTER // _CB
_NB_AD = (_E * _NR) // _CB
_NSTEPS = _NB_BASE + 2 * _NB_AD


def _mlp_body(idx_ref, scales_ref, x_ref, bg_ref, bu_ref, rg_ref, ru_ref,
              fg_ref, fu_ref, bd_ref, rd_ref, fd_ref, out_ref):
    c = pl.program_id(0)
    x = x_ref[...]

    def swiglu(g_w, u_w):
        dn = (((1,), (1,)), ((), ()))
        g = jax.lax.dot_general(x, g_w.astype(jnp.bfloat16), dn,
                                preferred_element_type=jnp.float32)
        u = jax.lax.dot_general(x, u_w.astype(jnp.bfloat16), dn,
                                preferred_element_type=jnp.float32)
        sig = 1.0 / (1.0 + jnp.exp(-g))
        return (g * sig) * u

    def accum(contrib):
        @pl.when(c == 0)
        def _():
            out_ref[...] = contrib

        @pl.when(c > 0)
        def _():
            out_ref[...] += contrib

    def adapter_mask(h, block_in_adapter, scale_col):
        col = block_in_adapter * _CB + jax.lax.broadcasted_iota(
            jnp.int32, (_T, _CB), 1)
        ecol = col // _NR
        idxv = idx_ref[...]
        sel = (ecol == idxv)
        e_iota = jax.lax.broadcasted_iota(jnp.int32, (_T, _E), 1)
        onehot = (e_iota == idxv).astype(jnp.float32)
        s = jnp.dot(onehot, scales_ref[:, scale_col:scale_col + 1],
                    preferred_element_type=jnp.float32)
        return jnp.where(sel, h * s, 0.0)

    @pl.when(c < _NB_BASE)
    def _():
        h = swiglu(bg_ref[...], bu_ref[...])
        dn = (((1,), (1,)), ((), ()))
        contrib = jax.lax.dot_general(
            h.astype(jnp.bfloat16), bd_ref[...].astype(jnp.bfloat16), dn,
            preferred_element_type=jnp.float32)
        accum(contrib)

    @pl.when((c >= _NB_BASE) & (c < _NB_BASE + _NB_AD))
    def _():
        h = swiglu(rg_ref[...], ru_ref[...])
        h = adapter_mask(h, c - _NB_BASE, 0)
        contrib = jnp.dot(h.astype(jnp.bfloat16), rd_ref[...],
                          preferred_element_type=jnp.float32)
        accum(contrib)

    @pl.when(c >= _NB_BASE + _NB_AD)
    def _():
        h = swiglu(fg_ref[...], fu_ref[...])
        h = adapter_mask(h, c - _NB_BASE - _NB_AD, 1)
        contrib = jnp.dot(h.astype(jnp.bfloat16), fd_ref[...],
                          preferred_element_type=jnp.float32)
        accum(contrib)


def kernel(x, token_lora_indices, base_gate_w, base_up_w, base_down_w,
           retain_gate_stacked, retain_up_stacked, retain_down_stacked,
           forget_gate_stacked, forget_up_stacked, forget_down_stacked,
           scales):
    idx = jnp.maximum(token_lora_indices, 0).reshape(_T, 1)
    xb = x.astype(jnp.bfloat16)
    rg = retain_gate_stacked.reshape(_E * _NR, _HID)
    ru = retain_up_stacked.reshape(_E * _NR, _HID)
    fg = forget_gate_stacked.reshape(_E * _NF, _HID)
    fu = forget_up_stacked.reshape(_E * _NF, _HID)
    rd = retain_down_stacked[:, 0].transpose(0, 2, 1).reshape(
        _E * _NR, _HID).astype(jnp.bfloat16)
    fd = forget_down_stacked[:, 0].transpose(0, 2, 1).reshape(
        _E * _NF, _HID).astype(jnp.bfloat16)

    nb = _NB_BASE
    na = _NB_AD

    def clamp(lo, hi):
        return lambda c: (jnp.clip(c - lo, 0, hi - 1), 0)

    grid_spec = dict(
        grid=(_NSTEPS,),
        in_specs=[
            pl.BlockSpec((_T, 1), lambda c: (0, 0)),
            pl.BlockSpec((_E, 2), lambda c: (0, 0)),
            pl.BlockSpec((_T, _HID), lambda c: (0, 0)),
            pl.BlockSpec((_CB, _HID), clamp(0, nb)),
            pl.BlockSpec((_CB, _HID), clamp(0, nb)),
            pl.BlockSpec((_CB, _HID), clamp(nb, na)),
            pl.BlockSpec((_CB, _HID), clamp(nb, na)),
            pl.BlockSpec((_CB, _HID), clamp(nb + na, na)),
            pl.BlockSpec((_CB, _HID), clamp(nb + na, na)),
            pl.BlockSpec((_HID, _CB),
                         lambda c: (0, jnp.clip(c, 0, nb - 1))),
            pl.BlockSpec((_CB, _HID), clamp(nb, na)),
            pl.BlockSpec((_CB, _HID), clamp(nb + na, na)),
        ],
        out_specs=pl.BlockSpec((_T, _HID), lambda c: (0, 0)),
    )

    out = pl.pallas_call(
        _mlp_body,
        **grid_spec,
        out_shape=jax.ShapeDtypeStruct((_T, _HID), jnp.float32),
        compiler_params=pltpu.CompilerParams(
            dimension_semantics=("arbitrary",)),
    )(idx, scales, xb, base_gate_w, base_up_w, rg, ru, fg, fu,
      base_down_w, rd, fd)
    return out
